# baseline (speedup 1.0000x reference)
import functools

import jax
import jax.numpy as jnp
from jax import lax
from jax.experimental import pallas as pl
from jax.experimental.pallas import tpu as pltpu
from jax.experimental.pallas import tpu_sc as plsc

_N_SLOW = 8
_N_FAST = 32
_BC = 12
_NC = 2
_NS = 16
_NW = _NC * _NS
_SLOW_PER_W = (_BC * _N_SLOW) // _NW
_FAST_PER_W = (_BC * _N_FAST) // _NW
_ROWS_PER_W = _SLOW_PER_W + _FAST_PER_W


def _sc_body(frames_hbm, slow_hbm, fast_hbm, sem):
    cid = lax.axis_index("c")
    sid = lax.axis_index("s")
    wid = sid * _NC + cid

    copies = []
    for j in range(_ROWS_PER_W):
        if j < _SLOW_PER_W:
            s = wid * _SLOW_PER_W + j
            bc = s // _N_SLOW
            t = s % _N_SLOW
            src = frames_hbm.at[bc, t * 9]
            dst = slow_hbm.at[bc, t]
        else:
            s = wid * _FAST_PER_W + (j - _SLOW_PER_W)
            bc = s // _N_FAST
            t = s % _N_FAST
            src = frames_hbm.at[bc, 2 * t + jnp.where(t == _N_FAST - 1, 1, 0)]
            dst = fast_hbm.at[bc, t]
        c = pltpu.make_async_copy(src, dst, sem)
        c.start()
        copies.append(c)
    for c in copies:
        c.wait()


def kernel(frames):
    B, C, T, H, W = frames.shape
    frames3d = frames.reshape(B * C, T, H, W)

    mesh = plsc.VectorSubcoreMesh(core_axis_name="c", subcore_axis_name="s",
                                  num_cores=_NC, num_subcores=_NS)
    run = functools.partial(
        pl.kernel,
        out_type=(
            jax.ShapeDtypeStruct((_BC, _N_SLOW, H, W), jnp.float32),
            jax.ShapeDtypeStruct((_BC, _N_FAST, H, W), jnp.float32),
        ),
        mesh=mesh,
        scratch_types=[
            pltpu.SemaphoreType.DMA,
        ],
        compiler_params=pltpu.CompilerParams(use_tc_tiling_on_sc=True),
    )(_sc_body)
    slow4d, fast4d = run(frames3d)
    slow = slow4d.reshape(B, C, _N_SLOW, H, W)
    fast = fast4d.reshape(B, C, _N_FAST, H, W)
    return (slow, fast)


# speedup vs baseline: 35.0461x; 35.0461x over previous
import functools

import jax
import jax.numpy as jnp
from jax import lax
from jax.experimental import pallas as pl
from jax.experimental.pallas import tpu as pltpu
from jax.experimental.pallas import tpu_sc as plsc

_N_SLOW = 8
_N_FAST = 32
_BC = 12
_NC = 2
_NS = 16
_NW = _NC * _NS
_SLOW_PER_W = (_BC * _N_SLOW) // _NW
_FAST_PER_W = (_BC * _N_FAST) // _NW
_ROWS_PER_W = _SLOW_PER_W + _FAST_PER_W


def _sc_body(frames_hbm, slow_hbm, fast_hbm,
             buf0, buf1, sem_in0, sem_in1, sem_out0, sem_out1):
    cid = lax.axis_index("c")
    sid = lax.axis_index("s")
    wid = sid * _NC + cid

    def plane(j):
        if j < _SLOW_PER_W:
            s = wid * _SLOW_PER_W + j
            bc = s // _N_SLOW
            t = s % _N_SLOW
            return frames_hbm.at[bc, t * 9], slow_hbm.at[bc, t]
        s = wid * _FAST_PER_W + (j - _SLOW_PER_W)
        bc = s // _N_FAST
        t = s % _N_FAST
        src_t = 2 * t + jnp.where(t == _N_FAST - 1, 1, 0)
        return frames_hbm.at[bc, src_t], fast_hbm.at[bc, t]

    bufs = (buf0, buf1)
    sems_in = (sem_in0, sem_in1)
    sems_out = (sem_out0, sem_out1)
    pending_wr = [None, None]
    pending_rd = [None, None]

    src0, _ = plane(0)
    pending_rd[0] = pltpu.make_async_copy(src0, buf0, sem_in0)
    pending_rd[0].start()
    for j in range(_ROWS_PER_W):
        k = j % 2
        kn = (j + 1) % 2
        if j + 1 < _ROWS_PER_W:
            if pending_wr[kn] is not None:
                pending_wr[kn].wait()
            src_n, _ = plane(j + 1)
            pending_rd[kn] = pltpu.make_async_copy(src_n, bufs[kn],
                                                   sems_in[kn])
            pending_rd[kn].start()
        pending_rd[k].wait()
        _, dst = plane(j)
        wr = pltpu.make_async_copy(bufs[k], dst, sems_out[k])
        wr.start()
        pending_wr[k] = wr
    pending_wr[0].wait()
    pending_wr[1].wait()


def kernel(frames):
    B, C, T, H, W = frames.shape
    frames3d = frames.reshape(B * C, T, H, W)

    mesh = plsc.VectorSubcoreMesh(core_axis_name="c", subcore_axis_name="s",
                                  num_cores=_NC, num_subcores=_NS)
    run = functools.partial(
        pl.kernel,
        out_type=(
            jax.ShapeDtypeStruct((_BC, _N_SLOW, H, W), jnp.float32),
            jax.ShapeDtypeStruct((_BC, _N_FAST, H, W), jnp.float32),
        ),
        mesh=mesh,
        scratch_types=[
            pltpu.VMEM((H, W), jnp.float32),
            pltpu.VMEM((H, W), jnp.float32),
            pltpu.SemaphoreType.DMA,
            pltpu.SemaphoreType.DMA,
            pltpu.SemaphoreType.DMA,
            pltpu.SemaphoreType.DMA,
        ],
        compiler_params=pltpu.CompilerParams(use_tc_tiling_on_sc=True),
    )(_sc_body)
    slow4d, fast4d = run(frames3d)
    slow = slow4d.reshape(B, C, _N_SLOW, H, W)
    fast = fast4d.reshape(B, C, _N_FAST, H, W)
    return (slow, fast)


# speedup vs baseline: 36.3629x; 1.0376x over previous
import functools

import jax
import jax.numpy as jnp
from jax import lax
from jax.experimental import pallas as pl
from jax.experimental.pallas import tpu as pltpu
from jax.experimental.pallas import tpu_sc as plsc

_N_SLOW = 8
_N_FAST = 32
_BC = 12
_NC = 2
_NS = 16
_NW = _NC * _NS
_FAST_PER_W = (_BC * _N_FAST) // _NW


def _sc_fast_body(frames_hbm, fast_hbm,
                  buf0, buf1, sem_in0, sem_in1, sem_out0, sem_out1):
    cid = lax.axis_index("c")
    sid = lax.axis_index("s")
    wid = sid * _NC + cid

    def plane(j):
        s = wid * _FAST_PER_W + j
        bc = s // _N_FAST
        t = s % _N_FAST
        src_t = 2 * t + jnp.where(t == _N_FAST - 1, 1, 0)
        return frames_hbm.at[bc, src_t], fast_hbm.at[bc, t]

    bufs = (buf0, buf1)
    sems_in = (sem_in0, sem_in1)
    sems_out = (sem_out0, sem_out1)
    pending_wr = [None, None]
    pending_rd = [None, None]

    src0, _ = plane(0)
    pending_rd[0] = pltpu.make_async_copy(src0, buf0, sem_in0)
    pending_rd[0].start()
    for j in range(_FAST_PER_W):
        k = j % 2
        kn = (j + 1) % 2
        if j + 1 < _FAST_PER_W:
            if pending_wr[kn] is not None:
                pending_wr[kn].wait()
            src_n, _ = plane(j + 1)
            pending_rd[kn] = pltpu.make_async_copy(src_n, bufs[kn],
                                                   sems_in[kn])
            pending_rd[kn].start()
        pending_rd[k].wait()
        _, dst = plane(j)
        wr = pltpu.make_async_copy(bufs[k], dst, sems_out[k])
        wr.start()
        pending_wr[k] = wr
    pending_wr[0].wait()
    pending_wr[1].wait()


def _tc_slow_body(frames_ref, slow_ref):
    slow_ref[...] = frames_ref[...]


def kernel(frames):
    B, C, T, H, W = frames.shape
    frames3d = frames.reshape(B * C, T, H, W)

    mesh = plsc.VectorSubcoreMesh(core_axis_name="c", subcore_axis_name="s",
                                  num_cores=_NC, num_subcores=_NS)
    run_fast = functools.partial(
        pl.kernel,
        out_type=jax.ShapeDtypeStruct((_BC, _N_FAST, H, W), jnp.float32),
        mesh=mesh,
        scratch_types=[
            pltpu.VMEM((H, W), jnp.float32),
            pltpu.VMEM((H, W), jnp.float32),
            pltpu.SemaphoreType.DMA,
            pltpu.SemaphoreType.DMA,
            pltpu.SemaphoreType.DMA,
            pltpu.SemaphoreType.DMA,
        ],
        compiler_params=pltpu.CompilerParams(use_tc_tiling_on_sc=True),
    )(_sc_fast_body)
    fast4d = run_fast(frames3d)

    slow4d = pl.pallas_call(
        _tc_slow_body,
        grid=(_N_SLOW,),
        in_specs=[pl.BlockSpec((_BC, 1, H, W), lambda t: (0, 9 * t, 0, 0))],
        out_specs=pl.BlockSpec((_BC, 1, H, W), lambda t: (0, t, 0, 0)),
        out_shape=jax.ShapeDtypeStruct((_BC, _N_SLOW, H, W), jnp.float32),
    )(frames3d)

    slow = slow4d.reshape(B, C, _N_SLOW, H, W)
    fast = fast4d.reshape(B, C, _N_FAST, H, W)
    return (slow, fast)


# speedup vs baseline: 36.9482x; 1.0161x over previous
import functools

import jax
import jax.numpy as jnp
from jax import lax
from jax.experimental import pallas as pl
from jax.experimental.pallas import tpu as pltpu
from jax.experimental.pallas import tpu_sc as plsc

_N_SLOW = 8
_N_FAST = 32
_BC = 12
_NC = 2
_NS = 16
_NW = _NC * _NS
_SLOW_PER_W = (_BC * _N_SLOW) // _NW


def _sc_slow_body(frames_hbm, slow_hbm,
                  buf0, buf1, sem_in0, sem_in1, sem_out0, sem_out1):
    cid = lax.axis_index("c")
    sid = lax.axis_index("s")
    wid = sid * _NC + cid

    def plane(j):
        s = wid * _SLOW_PER_W + j
        bc = s // _N_SLOW
        t = s % _N_SLOW
        return frames_hbm.at[bc, t * 9], slow_hbm.at[bc, t]

    bufs = (buf0, buf1)
    sems_in = (sem_in0, sem_in1)
    sems_out = (sem_out0, sem_out1)
    pending_wr = [None, None]
    pending_rd = [None, None]

    src0, _ = plane(0)
    pending_rd[0] = pltpu.make_async_copy(src0, buf0, sem_in0)
    pending_rd[0].start()
    for j in range(_SLOW_PER_W):
        k = j % 2
        kn = (j + 1) % 2
        if j + 1 < _SLOW_PER_W:
            if pending_wr[kn] is not None:
                pending_wr[kn].wait()
            src_n, _ = plane(j + 1)
            pending_rd[kn] = pltpu.make_async_copy(src_n, bufs[kn],
                                                   sems_in[kn])
            pending_rd[kn].start()
        pending_rd[k].wait()
        _, dst = plane(j)
        wr = pltpu.make_async_copy(bufs[k], dst, sems_out[k])
        wr.start()
        pending_wr[k] = wr
    pending_wr[0].wait()
    pending_wr[1].wait()


def _tc_fast_body(frames_ref, fast_ref):
    fast_ref[...] = frames_ref[...]


def kernel(frames):
    B, C, T, H, W = frames.shape
    frames3d = frames.reshape(B * C, T, H, W)

    mesh = plsc.VectorSubcoreMesh(core_axis_name="c", subcore_axis_name="s",
                                  num_cores=_NC, num_subcores=_NS)
    run_slow = functools.partial(
        pl.kernel,
        out_type=jax.ShapeDtypeStruct((_BC, _N_SLOW, H, W), jnp.float32),
        mesh=mesh,
        scratch_types=[
            pltpu.VMEM((H, W), jnp.float32),
            pltpu.VMEM((H, W), jnp.float32),
            pltpu.SemaphoreType.DMA,
            pltpu.SemaphoreType.DMA,
            pltpu.SemaphoreType.DMA,
            pltpu.SemaphoreType.DMA,
        ],
        compiler_params=pltpu.CompilerParams(use_tc_tiling_on_sc=True),
    )(_sc_slow_body)

    fast4d = pl.pallas_call(
        _tc_fast_body,
        grid=(_N_FAST,),
        in_specs=[pl.BlockSpec(
            (_BC, 1, H, W),
            lambda t: (0, 2 * t + jnp.where(t == _N_FAST - 1, 1, 0), 0, 0))],
        out_specs=pl.BlockSpec((_BC, 1, H, W), lambda t: (0, t, 0, 0)),
        out_shape=jax.ShapeDtypeStruct((_BC, _N_FAST, H, W), jnp.float32),
    )(frames3d)
    slow4d = run_slow(frames3d)

    slow = slow4d.reshape(B, C, _N_SLOW, H, W)
    fast = fast4d.reshape(B, C, _N_FAST, H, W)
    return (slow, fast)
